# baseline (speedup 1.0000x reference)
import functools

import jax
import jax.numpy as jnp
from jax import lax
from jax.experimental import pallas as pl
from jax.experimental.pallas import tpu as pltpu
from jax.experimental.pallas import tpu_sc as plsc

_N = 10000
_E = 320000
_D = 128
_OUT = 64

_NP = 10240
_ROWS_PER_SUB = _NP // 16
_CH = 128
_NTILES = 32
_NCH = 80
_NBLK = 5
_BCH = _NCH // _NBLK
_EPAD = _NTILES * _NCH * _CH


def _sc_aggregate(x, src3, dst3, zrow, zdeg, ones):
    mesh = plsc.VectorSubcoreMesh(core_axis_name="c", subcore_axis_name="s")

    @functools.partial(
        pl.kernel,
        mesh=mesh,
        out_type=[
            jax.ShapeDtypeStruct((2, _NP, _D), jnp.float32),
            jax.ShapeDtypeStruct((2, _NP), jnp.float32),
        ],
        scratch_types=[
            pltpu.VMEM((_BCH, _CH), jnp.int32),
            pltpu.VMEM((_BCH, _CH), jnp.int32),
            pltpu.VMEM((_BCH, _CH), jnp.int32),
            pltpu.VMEM((_BCH, _CH), jnp.int32),
            pltpu.VMEM((_CH, _D), jnp.float32),
            pltpu.VMEM((_CH, _D), jnp.float32),
            pltpu.VMEM((_CH,), jnp.float32),
            pltpu.VMEM_SHARED((_NP, _D), jnp.float32),
            pltpu.VMEM_SHARED((_NP,), jnp.float32),
            pltpu.SemaphoreType.DMA,
            pltpu.SemaphoreType.DMA,
            pltpu.SemaphoreType.DMA,
            pltpu.SemaphoreType.DMA,
            pltpu.SemaphoreType.DMA,
            pltpu.SemaphoreType.DMA,
            pltpu.SemaphoreType.DMA,
        ],
    )
    def k(x_hbm, src_hbm, dst_hbm, zrow_hbm, zdeg_hbm, ones_hbm,
          agg_out, deg_out, sA, dA, sB, dB, rows0, rows1, onesv, aggsh, degsh,
          sg0, sg1, ss0, ss1, sd0, sd1, si):
        c = lax.axis_index("c")
        s = lax.axis_index("s")
        w = c * 16 + s
        r0 = s * _ROWS_PER_SUB
        pltpu.sync_copy(zrow_hbm, aggsh.at[pl.ds(r0, _ROWS_PER_SUB)])
        pltpu.sync_copy(zdeg_hbm, degsh.at[pl.ds(r0, _ROWS_PER_SUB)])
        pltpu.sync_copy(ones_hbm, onesv)
        pltpu.sync_copy(src_hbm.at[w, pl.ds(0, _BCH)], sA)
        pltpu.sync_copy(dst_hbm.at[w, pl.ds(0, _BCH)], dA)
        plsc.subcore_barrier()
        pltpu.async_copy(x_hbm.at[sA.at[0]], rows0, sg0)

        npairs = _BCH // 2

        def make_body(cur_s, cur_d, prev_d, drain_first):
            def body(i, carry):
                a = 2 * i
                b = a + 1
                pltpu.make_async_copy(x_hbm.at[cur_s.at[a]], rows0, sg0).wait()

                if drain_first:
                    @pl.when(i == 0)
                    def _():
                        pltpu.make_async_copy(rows1,
                                              aggsh.at[prev_d.at[_BCH - 1]],
                                              ss1).wait()
                        pltpu.make_async_copy(onesv,
                                              degsh.at[prev_d.at[_BCH - 1]],
                                              sd1).wait()

                    @pl.when(i > 0)
                    def _():
                        pltpu.make_async_copy(rows1, aggsh.at[cur_d.at[a - 1]],
                                              ss1).wait()
                        pltpu.make_async_copy(onesv, degsh.at[cur_d.at[a - 1]],
                                              sd1).wait()
                else:
                    @pl.when(i > 0)
                    def _():
                        pltpu.make_async_copy(rows1, aggsh.at[cur_d.at[a - 1]],
                                              ss1).wait()
                        pltpu.make_async_copy(onesv, degsh.at[cur_d.at[a - 1]],
                                              sd1).wait()

                pltpu.async_copy(x_hbm.at[cur_s.at[b]], rows1, sg1)
                pltpu.async_copy(rows0, aggsh.at[cur_d.at[a]], ss0, add=True)
                pltpu.async_copy(onesv, degsh.at[cur_d.at[a]], sd0, add=True)
                pltpu.make_async_copy(x_hbm.at[cur_s.at[b]], rows1, sg1).wait()
                pltpu.make_async_copy(rows0, aggsh.at[cur_d.at[a]], ss0).wait()
                pltpu.make_async_copy(onesv, degsh.at[cur_d.at[a]], sd0).wait()

                @pl.when(i + 1 < npairs)
                def _():
                    pltpu.async_copy(x_hbm.at[cur_s.at[a + 2]], rows0, sg0)

                pltpu.async_copy(rows1, aggsh.at[cur_d.at[b]], ss1, add=True)
                pltpu.async_copy(onesv, degsh.at[cur_d.at[b]], sd1, add=True)
                return carry
            return body

        bufs = [(sA, dA), (sB, dB)]
        for blk in range(_NBLK):
            cur_s, cur_d = bufs[blk % 2]
            nxt_s, nxt_d = bufs[(blk + 1) % 2]
            prev_d = bufs[(blk + 1) % 2][1]
            if blk + 1 < _NBLK:
                pltpu.async_copy(src_hbm.at[w, pl.ds((blk + 1) * _BCH, _BCH)],
                                 nxt_s, si)
                pltpu.async_copy(dst_hbm.at[w, pl.ds((blk + 1) * _BCH, _BCH)],
                                 nxt_d, si)
            lax.fori_loop(0, npairs,
                          make_body(cur_s, cur_d, prev_d, drain_first=blk > 0),
                          0)
            if blk + 1 < _NBLK:
                pltpu.make_async_copy(src_hbm.at[w, pl.ds((blk + 1) * _BCH, _BCH)],
                                      nxt_s, si).wait()
                pltpu.make_async_copy(dst_hbm.at[w, pl.ds((blk + 1) * _BCH, _BCH)],
                                      nxt_d, si).wait()
                pltpu.async_copy(x_hbm.at[nxt_s.at[0]], rows0, sg0)

        last_d = bufs[(_NBLK - 1) % 2][1]
        pltpu.make_async_copy(rows1, aggsh.at[last_d.at[_BCH - 1]], ss1).wait()
        pltpu.make_async_copy(onesv, degsh.at[last_d.at[_BCH - 1]], sd1).wait()
        plsc.subcore_barrier()
        pltpu.sync_copy(aggsh.at[pl.ds(r0, _ROWS_PER_SUB)],
                        agg_out.at[c, pl.ds(r0, _ROWS_PER_SUB)])
        pltpu.sync_copy(degsh.at[pl.ds(r0, _ROWS_PER_SUB)],
                        deg_out.at[c, pl.ds(r0, _ROWS_PER_SUB)])

    return k(x, src3, dst3, zrow, zdeg, ones)


def _tc_body(x_ref, aggp_ref, degp_ref, eps_ref,
             w1_ref, b1_ref, g1_ref, be1_ref,
             w2_ref, b2_ref, g2_ref, be2_ref,
             wp0_ref, bp0_ref, wc_ref, bc_ref, out_ref):
    xv = x_ref[...]
    agg = aggp_ref[0, :_N, :] + aggp_ref[1, :_N, :]
    deg = degp_ref[0, :_N, :] + degp_ref[1, :_N, :]
    agg = agg / jnp.maximum(deg, 1.0)
    rst = (1.0 + eps_ref[0, 0]) * xv + agg
    t = jnp.dot(rst, w1_ref[...], preferred_element_type=jnp.float32) + b1_ref[...]
    mu = jnp.mean(t, axis=0, keepdims=True)
    var = jnp.mean((t - mu) ** 2, axis=0, keepdims=True)
    h = jnp.maximum((t - mu) / jnp.sqrt(var + 1e-5) * g1_ref[...] + be1_ref[...], 0.0)
    t2 = jnp.dot(h, w2_ref[...], preferred_element_type=jnp.float32) + b2_ref[...]
    mu2 = jnp.mean(t2, axis=0, keepdims=True)
    var2 = jnp.mean((t2 - mu2) ** 2, axis=0, keepdims=True)
    h2 = jnp.maximum((t2 - mu2) / jnp.sqrt(var2 + 1e-5) * g2_ref[...] + be2_ref[...], 0.0)
    s2 = jnp.sum(h2, axis=0, keepdims=True)
    sx = jnp.sum(xv, axis=0, keepdims=True)
    out0 = jnp.dot(sx, wp0_ref[...], preferred_element_type=jnp.float32) + bp0_ref[...]
    out1 = jnp.dot(s2, wc_ref[...], preferred_element_type=jnp.float32) + bc_ref[...]
    out_ref[...] = out0 + out1


def kernel(x, edge_index, eps0, W1, b1, g1, be1, W2, b2, g2, be2, Wp0, bp0, Wc, bc):
    pad = _EPAD - _E
    src3 = jnp.pad(edge_index[0], (0, pad)).reshape(_NTILES, _NCH, _CH)
    dst3 = jnp.pad(edge_index[1], (0, pad),
                   constant_values=_N).reshape(_NTILES, _NCH, _CH)
    zrow = jnp.zeros((_ROWS_PER_SUB, _D), jnp.float32)
    zdeg = jnp.zeros((_ROWS_PER_SUB,), jnp.float32)
    ones = jnp.ones((_CH,), jnp.float32)
    aggp, degp = _sc_aggregate(x, src3, dst3, zrow, zdeg, ones)

    out = pl.pallas_call(
        _tc_body,
        out_shape=jax.ShapeDtypeStruct((1, _OUT), jnp.float32),
    )(x, aggp, degp.reshape(2, _NP, 1), eps0.reshape(1, 1),
      W1, b1.reshape(1, _D), g1.reshape(1, _D), be1.reshape(1, _D),
      W2, b2.reshape(1, _D), g2.reshape(1, _D), be2.reshape(1, _D),
      Wp0, bp0.reshape(1, _OUT), Wc, bc.reshape(1, _OUT))
    return out
